# Initial kernel scaffold; baseline (speedup 1.0000x reference)
import jax
import jax.numpy as jnp
from jax.experimental import pallas as pl


def kernel(x, gate_w):
    raise NotImplementedError("write your pallas kernel here")


# speedup vs baseline: 1.0056x; 1.0056x over previous
import functools

import jax
import jax.numpy as jnp
from jax.experimental import pallas as pl
from jax.experimental.pallas import tpu as pltpu

_TOP_K = 8


def _router_block(x_ref, wt_ref, scores_ref, idx_ref):
    logits = jnp.dot(x_ref[...], wt_ref[...],
                     preferred_element_type=jnp.float32)
    b, e = logits.shape
    col = jax.lax.broadcasted_iota(jnp.int32, (b, e), 1)
    vals = logits
    top_vals, top_idx = [], []
    for _ in range(_TOP_K):
        m = jnp.max(vals, axis=1, keepdims=True)
        idx = jnp.min(jnp.where(vals == m, col, e), axis=1, keepdims=True)
        top_vals.append(m)
        top_idx.append(idx)
        vals = jnp.where(col == idx, -jnp.inf, vals)
    tv = jnp.concatenate(top_vals, axis=1)
    ti = jnp.concatenate(top_idx, axis=1)
    ex = jnp.exp(tv - tv[:, :1])
    scores_ref[...] = ex / jnp.sum(ex, axis=1, keepdims=True)
    idx_ref[...] = ti


@jax.jit
def kernel(x, gate_w):
    tokens, dim = x.shape
    n_exp = gate_w.shape[0]
    wt = gate_w.T
    blk = 512
    scores, idx = pl.pallas_call(
        _router_block,
        grid=(tokens // blk,),
        in_specs=[
            pl.BlockSpec((blk, dim), lambda i: (i, 0)),
            pl.BlockSpec((dim, n_exp), lambda i: (0, 0)),
        ],
        out_specs=[
            pl.BlockSpec((blk, _TOP_K), lambda i: (i, 0)),
            pl.BlockSpec((blk, _TOP_K), lambda i: (i, 0)),
        ],
        out_shape=[
            jax.ShapeDtypeStruct((tokens, _TOP_K), jnp.float32),
            jax.ShapeDtypeStruct((tokens, _TOP_K), jnp.int32),
        ],
    )(x, wt)
    return scores, idx
